# Initial kernel scaffold; baseline (speedup 1.0000x reference)
import jax
import jax.numpy as jnp
from jax.experimental import pallas as pl


def kernel(input, support, W):
    raise NotImplementedError("write your pallas kernel here")


# speedup vs baseline: 1.4576x; 1.4576x over previous
import functools

import jax
import jax.numpy as jnp
from jax import lax
from jax.experimental import pallas as pl
from jax.experimental.pallas import tpu as pltpu
from jax.experimental.pallas import tpu_sc as plsc

D = 32
B_IN = 16384 * 50
B_SUP = 16384 * 20
NC, NS = 2, 16
NW = NC * NS
CHUNK = 1024

_mesh = plsc.VectorSubcoreMesh(core_axis_name="c", subcore_axis_name="s")


@functools.partial(
    pl.kernel,
    out_type=(
        jax.ShapeDtypeStruct((B_IN, D), jnp.float32),
        jax.ShapeDtypeStruct((B_SUP, D), jnp.float32),
    ),
    mesh=_mesh,
    scratch_types=[
        pltpu.VMEM((CHUNK,), jnp.int32),
        pltpu.VMEM((CHUNK, D), jnp.float32),
        pltpu.SemaphoreType.DMA,
    ],
    compiler_params=pltpu.CompilerParams(use_tc_tiling_on_sc=False),
)
def _emb_lookup(in_idx, sup_idx, table, out_in, out_sup, idx_v, rows_v, sem):
    wid = lax.axis_index("s") * NC + lax.axis_index("c")

    def make_body(idx_hbm, out_hbm, rows_per_w):
        base_w = wid * rows_per_w

        def body(i, carry):
            base = base_w + i * CHUNK
            pltpu.sync_copy(idx_hbm.at[pl.ds(base, CHUNK)], idx_v)
            pltpu.async_copy(table.at[idx_v], rows_v, sem).wait()
            pltpu.sync_copy(rows_v, out_hbm.at[pl.ds(base, CHUNK)])
            return carry

        return body

    lax.fori_loop(0, (B_IN // NW) // CHUNK, make_body(in_idx, out_in, B_IN // NW), 0)
    lax.fori_loop(0, (B_SUP // NW) // CHUNK, make_body(sup_idx, out_sup, B_SUP // NW), 0)


def kernel(input, support, W):
    out_in, out_sup = _emb_lookup(input.reshape(-1), support.reshape(-1), W)
    return (out_in.reshape(input.shape + (D,)),
            out_sup.reshape(support.shape + (D,)))


# speedup vs baseline: 1.4875x; 1.0205x over previous
import functools

import jax
import jax.numpy as jnp
from jax import lax
from jax.experimental import pallas as pl
from jax.experimental.pallas import tpu as pltpu
from jax.experimental.pallas import tpu_sc as plsc

D = 32
B_IN = 16384 * 50
B_SUP = 16384 * 20
NC, NS = 2, 16
NW = NC * NS
CHUNK = 1280
NBUF = 2

_mesh = plsc.VectorSubcoreMesh(core_axis_name="c", subcore_axis_name="s")


@functools.partial(
    pl.kernel,
    out_type=(
        jax.ShapeDtypeStruct((B_IN, D), jnp.float32),
        jax.ShapeDtypeStruct((B_SUP, D), jnp.float32),
    ),
    mesh=_mesh,
    scratch_types=[
        pltpu.VMEM((CHUNK,), jnp.int32),
        pltpu.VMEM((CHUNK,), jnp.int32),
        pltpu.VMEM((CHUNK, D), jnp.float32),
        pltpu.VMEM((CHUNK, D), jnp.float32),
        pltpu.SemaphoreType.DMA((NBUF,)),
        pltpu.SemaphoreType.DMA((NBUF,)),
        pltpu.SemaphoreType.DMA((NBUF,)),
    ],
    compiler_params=pltpu.CompilerParams(use_tc_tiling_on_sc=False),
)
def _emb_lookup(in_idx, sup_idx, table, out_in, out_sup,
                idx_v0, idx_v1, rows_v0, rows_v1, sem_idx, sem_g, sem_wb):
    wid = lax.axis_index("s") * NC + lax.axis_index("c")
    idx_bufs = (idx_v0, idx_v1)
    row_bufs = (rows_v0, rows_v1)

    def run(idx_hbm, out_hbm, rows_per_w):
        base_w = wid * rows_per_w
        nch = rows_per_w // CHUNK

        for b in range(NBUF):
            pltpu.async_copy(idx_hbm.at[pl.ds(base_w + b * CHUNK, CHUNK)],
                             idx_bufs[b], sem_idx.at[b])

        def pair_body(p, carry):
            for b in range(NBUF):
                c = p * NBUF + b
                base = base_w + c * CHUNK
                pltpu.make_async_copy(
                    idx_hbm.at[pl.ds(base, CHUNK)], idx_bufs[b],
                    sem_idx.at[b]).wait()

                @pl.when(p > 0)
                def _():
                    pltpu.make_async_copy(
                        row_bufs[b], out_hbm.at[pl.ds(base, CHUNK)],
                        sem_wb.at[b]).wait()

                pltpu.async_copy(table.at[idx_bufs[b]], row_bufs[b],
                                 sem_g.at[b]).wait()
                pltpu.async_copy(row_bufs[b], out_hbm.at[pl.ds(base, CHUNK)],
                                 sem_wb.at[b])

                @pl.when(c + NBUF < nch)
                def _():
                    pltpu.async_copy(
                        idx_hbm.at[pl.ds(base + NBUF * CHUNK, CHUNK)],
                        idx_bufs[b], sem_idx.at[b])

            return carry

        lax.fori_loop(0, nch // NBUF, pair_body, 0)
        for b in range(NBUF):
            pltpu.make_async_copy(
                row_bufs[b], out_hbm.at[pl.ds(base_w, CHUNK)],
                sem_wb.at[b]).wait()

    run(in_idx, out_in, B_IN // NW)
    run(sup_idx, out_sup, B_SUP // NW)


def kernel(input, support, W):
    out_in, out_sup = _emb_lookup(input.reshape(-1), support.reshape(-1), W)
    return (out_in.reshape(input.shape + (D,)),
            out_sup.reshape(support.shape + (D,)))


# speedup vs baseline: 1.4909x; 1.0023x over previous
import functools

import jax
import jax.numpy as jnp
from jax import lax
from jax.experimental import pallas as pl
from jax.experimental.pallas import tpu as pltpu
from jax.experimental.pallas import tpu_sc as plsc

D = 32
B_IN = 16384 * 50
B_SUP = 16384 * 20
NC, NS = 2, 16
NW = NC * NS
CHUNK = 640
NBUF = 4

_mesh = plsc.VectorSubcoreMesh(core_axis_name="c", subcore_axis_name="s")


@functools.partial(
    pl.kernel,
    out_type=(
        jax.ShapeDtypeStruct((B_IN, D), jnp.float32),
        jax.ShapeDtypeStruct((B_SUP, D), jnp.float32),
    ),
    mesh=_mesh,
    scratch_types=(
        [pltpu.VMEM((CHUNK,), jnp.int32) for _ in range(NBUF)]
        + [pltpu.VMEM((CHUNK, D), jnp.float32) for _ in range(NBUF)]
        + [pltpu.SemaphoreType.DMA((NBUF,)),
           pltpu.SemaphoreType.DMA((NBUF,)),
           pltpu.SemaphoreType.DMA((NBUF,))]
    ),
    compiler_params=pltpu.CompilerParams(use_tc_tiling_on_sc=False),
)
def _emb_lookup(in_idx, sup_idx, table, out_in, out_sup, *scratch):
    idx_bufs = scratch[:NBUF]
    row_bufs = scratch[NBUF:2 * NBUF]
    sem_idx, sem_g, sem_wb = scratch[2 * NBUF:]
    wid = lax.axis_index("s") * NC + lax.axis_index("c")

    def run(idx_hbm, out_hbm, rows_per_w):
        base_w = wid * rows_per_w
        nch = rows_per_w // CHUNK

        for b in range(NBUF):
            pltpu.async_copy(idx_hbm.at[pl.ds(base_w + b * CHUNK, CHUNK)],
                             idx_bufs[b], sem_idx.at[b])

        def group_body(p, carry):
            gbase = base_w + p * NBUF * CHUNK
            for b in range(NBUF):
                base = gbase + b * CHUNK
                pltpu.make_async_copy(
                    idx_hbm.at[pl.ds(base, CHUNK)], idx_bufs[b],
                    sem_idx.at[b]).wait()

                @pl.when(p > 0)
                def _():
                    pltpu.make_async_copy(
                        row_bufs[b], out_hbm.at[pl.ds(base, CHUNK)],
                        sem_wb.at[b]).wait()

                pltpu.async_copy(table.at[idx_bufs[b]], row_bufs[b],
                                 sem_g.at[b])
            for b in range(NBUF):
                base = gbase + b * CHUNK
                pltpu.make_async_copy(table.at[idx_bufs[b]], row_bufs[b],
                                      sem_g.at[b]).wait()
                pltpu.async_copy(row_bufs[b], out_hbm.at[pl.ds(base, CHUNK)],
                                 sem_wb.at[b])

                @pl.when(p + 1 < nch // NBUF)
                def _():
                    pltpu.async_copy(
                        idx_hbm.at[pl.ds(base + NBUF * CHUNK, CHUNK)],
                        idx_bufs[b], sem_idx.at[b])

            return carry

        lax.fori_loop(0, nch // NBUF, group_body, 0)
        for b in range(NBUF):
            pltpu.make_async_copy(
                row_bufs[b], out_hbm.at[pl.ds(base_w, CHUNK)],
                sem_wb.at[b]).wait()

    run(in_idx, out_in, B_IN // NW)
    run(sup_idx, out_sup, B_SUP // NW)


def kernel(input, support, W):
    out_in, out_sup = _emb_lookup(input.reshape(-1), support.reshape(-1), W)
    return (out_in.reshape(input.shape + (D,)),
            out_sup.reshape(support.shape + (D,)))
